# Initial kernel scaffold; baseline (speedup 1.0000x reference)
import jax
import jax.numpy as jnp
from jax.experimental import pallas as pl


def kernel(x0, conv1_w, conv2_w, conv3_w, conv4_w, conv5_w, conv1_s, conv2_s, conv3_s, conv4_s, conv5_s, bn1_g, bn1_b, bn2_g, bn2_b, bn3_g, bn3_b, bn4_g, bn4_b, bn5_g, bn5_b, lin1_w, lin2_w, lin3_w, lbn1_g, lbn1_b, lbn2_g, lbn2_b, lbn3_g, lbn3_b):
    raise NotImplementedError("write your pallas kernel here")


# speedup vs baseline: 7.3331x; 7.3331x over previous
import functools

import jax
import jax.numpy as jnp
from jax import lax
from jax.experimental import pallas as pl
from jax.experimental.pallas import tpu as pltpu
from jax.experimental.pallas import tpu_sc as plsc

B = 8
N = 1024
K = 20
EMB = 1024
OUT = 40
BN_TOKENS = B * N
CP = 128
NW = 32
ROWS = K * BN_TOKENS
ROWS_PER_W = ROWS // NW
NCHUNK = 512


def _knn_tc_body(xt_ref, idx_ref):
    b = pl.program_id(0)
    xt = xt_ref[0]
    g = lax.dot_general(xt, xt, (((1,), (1,)), ((), ())),
                        preferred_element_type=jnp.float32)
    inner = -2.0 * g
    xx = jnp.sum(xt * xt, axis=1)
    neg = (-xx[:, None]) - inner
    neg = neg - xx[None, :]

    iota = lax.broadcasted_iota(jnp.int32, (N, N), 1)
    base = b * N
    vals = neg
    for k in range(K):
        mx = jnp.max(vals, axis=1, keepdims=True)
        cand = jnp.where(vals == mx, iota, N)
        a = jnp.min(cand, axis=1)
        idx_ref[0, k, :] = a + base
        vals = jnp.where(iota == a[:, None], -jnp.inf, vals)


def _knn_tc(xt):
    C = xt.shape[2]
    return pl.pallas_call(
        _knn_tc_body,
        grid=(B,),
        in_specs=[pl.BlockSpec((1, N, C), lambda b: (b, 0, 0))],
        out_specs=pl.BlockSpec((1, K, N), lambda b: (b, 0, 0)),
        out_shape=jax.ShapeDtypeStruct((B, K, N), jnp.int32),
    )(xt)


def _make_gather_sc(R):
    n_chunks = ROWS_PER_W // R
    mesh = plsc.VectorSubcoreMesh(core_axis_name="c", subcore_axis_name="s")

    @functools.partial(
        pl.kernel,
        out_type=jax.ShapeDtypeStruct((ROWS, CP), jnp.float32),
        mesh=mesh,
        scratch_types=[
            pltpu.VMEM((ROWS_PER_W,), jnp.int32),
            pltpu.VMEM((R, CP), jnp.float32),
            pltpu.SemaphoreType.DMA,
        ],
    )
    def kern(table_hbm, idx_hbm, out_hbm, idx_v, rows_v, sem):
        wid = lax.axis_index("s") * 2 + lax.axis_index("c")
        rbase = wid * ROWS_PER_W
        pltpu.sync_copy(idx_hbm.at[pl.ds(rbase, ROWS_PER_W)], idx_v)

        @pl.loop(0, n_chunks)
        def _chunk(ci):
            pltpu.async_copy(
                table_hbm.at[idx_v.at[pl.ds(ci * R, R)]], rows_v, sem).wait()
            pltpu.sync_copy(rows_v, out_hbm.at[pl.ds(rbase + ci * R, R)])

    return kern


def _conv_max_body(nbr_ref, xt_ref, w_ref, s_ref, o_ref, *, C, pad):
    k = pl.program_id(2)
    ctr = xt_ref[0]
    nbr = nbr_ref[0, 0][:, :C]
    parts = [nbr - ctr, ctr]
    if pad:
        parts.append(jnp.zeros((NCHUNK, pad), jnp.float32))
    feat = jnp.concatenate(parts, axis=1)
    y = jnp.dot(feat, w_ref[...], preferred_element_type=jnp.float32)
    y = y * s_ref[...]

    @pl.when(k == 0)
    def _init():
        o_ref[0] = y

    @pl.when(k > 0)
    def _acc():
        o_ref[0] = jnp.maximum(o_ref[0], y)


def _conv_max_tc(nbr, xt, w2, s_row):
    C = xt.shape[2]
    O = w2.shape[1]
    pad = w2.shape[0] - 2 * C
    nc = N // NCHUNK
    return pl.pallas_call(
        functools.partial(_conv_max_body, C=C, pad=pad),
        grid=(B, nc, K),
        in_specs=[
            pl.BlockSpec((1, 1, NCHUNK, CP), lambda b, n, k: (k, b, n, 0)),
            pl.BlockSpec((1, NCHUNK, C), lambda b, n, k: (b, n, 0)),
            pl.BlockSpec(w2.shape, lambda b, n, k: (0, 0)),
            pl.BlockSpec((1, O), lambda b, n, k: (0, 0)),
        ],
        out_specs=pl.BlockSpec((1, NCHUNK, O), lambda b, n, k: (b, n, 0)),
        out_shape=jax.ShapeDtypeStruct((B, N, O), jnp.float32),
    )(nbr, xt, w2, s_row)


def _transpose_body(t_ref, o_ref):
    o_ref[0] = t_ref[0].T


def _transpose_tc(t):
    O = t.shape[2]
    return pl.pallas_call(
        _transpose_body,
        grid=(B,),
        in_specs=[pl.BlockSpec((1, N, O), lambda b: (b, 0, 0))],
        out_specs=pl.BlockSpec((1, O, N), lambda b: (b, 0, 0)),
        out_shape=jax.ShapeDtypeStruct((B, O, N), jnp.float32),
    )(t)


def _bn_tc(t, g, b):
    tt = _transpose_tc(t)
    m = jnp.mean(tt, axis=(0, 2), keepdims=True)
    v = jnp.var(tt, axis=(0, 2), keepdims=True)
    x = g[None, :, None] * (tt - m) * lax.rsqrt(v + 1e-5) + b[None, :, None]
    return jnp.transpose(x, (0, 2, 1))


def _conv5_stats_body(xc_ref, w_ref, sum_ref, sq_ref, mx_ref, mn_ref):
    x5 = jnp.dot(xc_ref[0], w_ref[...],
                 preferred_element_type=jnp.float32)
    sum_ref[0, 0, :] = jnp.sum(x5, axis=0)
    sq_ref[0, 0, :] = jnp.sum(x5 * x5, axis=0)
    mx_ref[0, 0, :] = jnp.max(x5, axis=0)
    mn_ref[0, 0, :] = jnp.min(x5, axis=0)


def _conv5_stats_tc(xc, w5t):
    return pl.pallas_call(
        _conv5_stats_body,
        grid=(B,),
        in_specs=[
            pl.BlockSpec((1, N, 512), lambda b: (b, 0, 0)),
            pl.BlockSpec((512, EMB), lambda b: (0, 0)),
        ],
        out_specs=[pl.BlockSpec((1, 1, EMB), lambda b: (b, 0, 0))] * 4,
        out_shape=[jax.ShapeDtypeStruct((B, 1, EMB), jnp.float32)] * 4,
    )(xc, w5t)


def _head_body(sum_ref, sq_ref, mx_ref, mn_ref, g5_ref, b5_ref,
               w1_ref, w2_ref, w3_ref,
               g1_ref, b1_ref, g2_ref, b2_ref, g3_ref, b3_ref, o_ref):
    s = sum_ref[...]
    q = sq_ref[...]
    tot = jnp.sum(s, axis=0, keepdims=True) / jnp.float32(BN_TOKENS)
    var = jnp.sum(q, axis=0, keepdims=True) / jnp.float32(BN_TOKENS) - tot * tot
    a5 = g5_ref[...] * lax.rsqrt(var + 1e-5)
    off = b5_ref[...] - a5 * tot
    hmax = jnp.where(a5 >= 0.0, a5 * mx_ref[...], a5 * mn_ref[...]) + off
    hmean = a5 * (s / jnp.float32(N)) + off
    h = jnp.concatenate([hmax, hmean], axis=1)

    def lin_bn(x, w_ref, g_ref, b_ref):
        y = jnp.dot(x, w_ref[...], preferred_element_type=jnp.float32)
        m = jnp.mean(y, axis=0, keepdims=True)
        v = jnp.mean((y - m) ** 2, axis=0, keepdims=True)
        return g_ref[...] * (y - m) * lax.rsqrt(v + 1e-5) + b_ref[...]

    h = lin_bn(h, w1_ref, g1_ref, b1_ref)
    h = lin_bn(h, w2_ref, g2_ref, b2_ref)
    o_ref[...] = lin_bn(h, w3_ref, g3_ref, b3_ref)


def _head_tc(stats, g5, b5, w1t, w2t, w3t, lbn):
    (g1, b1, g2, b2, g3, b3) = lbn
    return pl.pallas_call(
        _head_body,
        out_shape=jax.ShapeDtypeStruct((B, OUT), jnp.float32),
    )(stats[0].reshape(B, EMB), stats[1].reshape(B, EMB),
      stats[2].reshape(B, EMB), stats[3].reshape(B, EMB),
      g5.reshape(1, EMB), b5.reshape(1, EMB), w1t, w2t, w3t,
      g1.reshape(1, 512), b1.reshape(1, 512),
      g2.reshape(1, 256), b2.reshape(1, 256),
      g3.reshape(1, OUT), b3.reshape(1, OUT))


def _edge_layer(xt, conv_w, conv_s, bn_g, bn_b, gather_kern, dst):
    C = xt.shape[2]
    O = conv_w.shape[0]
    s_row = conv_s.reshape(1, O)

    xt_d = jnp.pad(xt, ((0, 0), (0, 0), (0, 5))) if C == 3 else xt
    idx_g = _knn_tc(xt_d)

    idx_km = jnp.transpose(idx_g, (1, 0, 2)).reshape(ROWS)
    table = jnp.pad(xt, ((0, 0), (0, 0), (0, CP - C))).reshape(BN_TOKENS, CP)
    nbr = gather_kern(table, idx_km).reshape(K, B, N, CP)

    w2 = conv_w.T
    if C == 3:
        w2 = jnp.pad(w2, ((0, 2), (0, 0)))
    t = _conv_max_tc(nbr, xt, w2, s_row)
    x_next = _bn_tc(t, bn_g, bn_b)

    src = jnp.transpose(idx_g, (0, 2, 1)).reshape(-1)
    ei = jnp.stack([src, dst])
    return x_next, ei


def kernel(x0, conv1_w, conv2_w, conv3_w, conv4_w, conv5_w,
           conv1_s, conv2_s, conv3_s, conv4_s, conv5_s,
           bn1_g, bn1_b, bn2_g, bn2_b, bn3_g, bn3_b, bn4_g, bn4_b, bn5_g, bn5_b,
           lin1_w, lin2_w, lin3_w,
           lbn1_g, lbn1_b, lbn2_g, lbn2_b, lbn3_g, lbn3_b):
    dst = jnp.repeat(jnp.arange(BN_TOKENS, dtype=jnp.int32), K)
    xt0 = jnp.transpose(x0, (0, 2, 1))

    gather = _make_gather_sc(320)

    x1, ei0 = _edge_layer(xt0, conv1_w, conv1_s, bn1_g, bn1_b, gather, dst)
    x2, ei1 = _edge_layer(x1, conv2_w, conv2_s, bn2_g, bn2_b, gather, dst)
    x3, ei2 = _edge_layer(x2, conv3_w, conv3_s, bn3_g, bn3_b, gather, dst)
    x4, ei3 = _edge_layer(x3, conv4_w, conv4_s, bn4_g, bn4_b, gather, dst)

    xc = jnp.concatenate([x1, x2, x3, x4], axis=2)
    w5t = conv5_w.T * conv5_s.reshape(EMB)[None, :]
    stats = _conv5_stats_tc(xc, w5t)
    logits = _head_tc(stats, bn5_g, bn5_b, lin1_w.T, lin2_w.T, lin3_w.T,
                      (lbn1_g, lbn1_b, lbn2_g, lbn2_b, lbn3_g, lbn3_b))

    aux0 = xt0.reshape(BN_TOKENS, 3)
    aux1 = x1.reshape(BN_TOKENS, 64)
    aux2 = x2.reshape(BN_TOKENS, 64)
    aux3 = x3.reshape(BN_TOKENS, 128)
    return (logits, ((aux0, ei0), (aux1, ei1), (aux2, ei2), (aux3, ei3)))
